# baseline (speedup 1.0000x reference)
import functools

import jax
import jax.numpy as jnp
from jax import lax
from jax.experimental import pallas as pl
from jax.experimental.pallas import tpu as pltpu
from jax.experimental.pallas import tpu_sc as plsc

NUM_FIELDS = 10
VOCAB = 100000
EMB_DIM = 25
BATCH = 4096
N_CONT = 13
N_EMB = NUM_FIELDS * EMB_DIM
HIDDEN = 64

NC, NS = 2, 16
NW = NC * NS
TOTAL_ROWS = BATCH * NUM_FIELDS
ROWS_PER_W = TOTAL_ROWS // NW
CHUNK = 128
CHUNKS_PER_W = ROWS_PER_W // CHUNK


@functools.partial(
    pl.kernel,
    mesh=plsc.VectorSubcoreMesh(core_axis_name="c", subcore_axis_name="s"),
    compiler_params=pltpu.CompilerParams(use_tc_tiling_on_sc=False),
    out_type=jax.ShapeDtypeStruct((NW, CHUNKS_PER_W, CHUNK, 32), jnp.float32),
    scratch_types=[
        pltpu.VMEM((CHUNKS_PER_W, CHUNK), jnp.int32),
        pltpu.VMEM((CHUNKS_PER_W, CHUNK, 32), jnp.float32),
        pltpu.SemaphoreType.DMA,
    ],
)
def _gather_sc(table_hbm, idx_hbm, out_hbm, idx_v, rows_v, sem):
    wid = lax.axis_index("s") * NC + lax.axis_index("c")
    pltpu.sync_copy(idx_hbm.at[wid], idx_v)
    copies = [
        pltpu.async_copy(
            table_hbm.at[idx_v.at[j]],
            rows_v.at[j],
            sem,
        )
        for j in range(CHUNKS_PER_W)
    ]
    for c in copies:
        c.wait()
    pltpu.sync_copy(rows_v, out_hbm.at[wid])


def _mlp_body(xe_ref, xc_ref, w1_ref, w2_ref, b_ref, o_ref):
    acc = jnp.dot(xe_ref[...], w1_ref[...], preferred_element_type=jnp.float32)
    acc = acc + jnp.dot(xc_ref[...], w2_ref[...], preferred_element_type=jnp.float32)
    o_ref[...] = jnp.maximum(acc + b_ref[...], 0.0)


def kernel(x_server_categorical, x_server_continuous, emb_tables, W, b):
    idx32 = x_server_categorical.astype(jnp.int32)
    flat_idx = idx32 + jnp.arange(NUM_FIELDS, dtype=jnp.int32) * VOCAB
    idx_3d = flat_idx.reshape(NW, CHUNKS_PER_W, CHUNK)
    table_flat = emb_tables.reshape(NUM_FIELDS * VOCAB, EMB_DIM)
    table_pad = jnp.pad(table_flat, ((0, 0), (0, 32 - EMB_DIM)))

    gathered = _gather_sc(table_pad, idx_3d)
    x_emb = gathered[..., :EMB_DIM].reshape(BATCH, N_EMB)

    W1 = W[:N_EMB]
    W2 = W[N_EMB:]
    b2d = b.reshape(1, HIDDEN)

    blk = 1024
    grid = BATCH // blk
    out = pl.pallas_call(
        _mlp_body,
        out_shape=jax.ShapeDtypeStruct((BATCH, HIDDEN), jnp.float32),
        grid=(grid,),
        in_specs=[
            pl.BlockSpec((blk, N_EMB), lambda i: (i, 0)),
            pl.BlockSpec((blk, N_CONT), lambda i: (i, 0)),
            pl.BlockSpec((N_EMB, HIDDEN), lambda i: (0, 0)),
            pl.BlockSpec((N_CONT, HIDDEN), lambda i: (0, 0)),
            pl.BlockSpec((1, HIDDEN), lambda i: (0, 0)),
        ],
        out_specs=pl.BlockSpec((blk, HIDDEN), lambda i: (i, 0)),
    )(x_emb, x_server_continuous, W1, W2, b2d)
    return out


# speedup vs baseline: 1.6769x; 1.6769x over previous
import functools

import jax
import jax.numpy as jnp
from jax import lax
from jax.experimental import pallas as pl
from jax.experimental.pallas import tpu as pltpu
from jax.experimental.pallas import tpu_sc as plsc

NUM_FIELDS = 10
VOCAB = 100000
EMB_DIM = 25
BATCH = 4096
N_CONT = 13
N_EMB = NUM_FIELDS * EMB_DIM
HIDDEN = 64
N_ROWS = NUM_FIELDS * VOCAB

NC, NS = 2, 16
NW = NC * NS
B_PER_W = BATCH // NW
SUB = 16
SUBS_PER_ROW = B_PER_W // SUB
NBUF = 3
FIELD_W = 32
PAD_D = 128


def _pad_body(t_ref, o_ref):
    o_ref[...] = jnp.pad(t_ref[...], ((0, 0), (0, PAD_D - EMB_DIM)))


def _pad_table(table):
    rows_blk = 8192
    return pl.pallas_call(
        _pad_body,
        out_shape=jax.ShapeDtypeStruct((N_ROWS, PAD_D), jnp.float32),
        grid=(N_ROWS // rows_blk,),
        in_specs=[pl.BlockSpec((rows_blk, EMB_DIM), lambda i: (i, 0))],
        out_specs=pl.BlockSpec((rows_blk, PAD_D), lambda i: (i, 0)),
    )(table)


@functools.partial(
    pl.kernel,
    mesh=plsc.VectorSubcoreMesh(core_axis_name="c", subcore_axis_name="s"),
    out_type=jax.ShapeDtypeStruct((BATCH, NUM_FIELDS * FIELD_W), jnp.float32),
    scratch_types=[
        pltpu.VMEM((NUM_FIELDS, B_PER_W), jnp.int32),
        pltpu.VMEM((NBUF, SUB, PAD_D), jnp.float32),
        pltpu.VMEM((B_PER_W, NUM_FIELDS * FIELD_W), jnp.float32),
        [pltpu.SemaphoreType.DMA] * NBUF,
    ],
)
def _gather_sc(table_hbm, idx_hbm, out_hbm, idx_v, rows_v, out_v, sems):
    wid = lax.axis_index("s") * NC + lax.axis_index("c")
    pltpu.sync_copy(idx_hbm.at[wid], idx_v)

    def do_subwave(i, s, buf):
        col0 = i * FIELD_W
        for k in range(SUB):
            out_v[SUB * s + k, pl.ds(col0, 16)] = rows_v[buf, k, pl.ds(0, 16)]
            out_v[SUB * s + k, pl.ds(col0 + 16, 16)] = rows_v[buf, k, pl.ds(16, 16)]

    def start_gather(i, s, buf):
        tvec = idx_v[i, pl.ds(SUB * s, SUB)]
        return pltpu.async_copy(table_hbm.at[tvec], rows_v.at[buf], sems[buf])

    def field_row(i, _):
        copies = []
        for s in range(SUBS_PER_ROW):
            copies.append(start_gather(i, s, s % NBUF))
            if s >= NBUF - 1:
                copies[s - (NBUF - 1)].wait()
                do_subwave(i, s - (NBUF - 1), (s - (NBUF - 1)) % NBUF)
        for s in range(SUBS_PER_ROW - (NBUF - 1), SUBS_PER_ROW):
            copies[s].wait()
            do_subwave(i, s, s % NBUF)
        return 0

    lax.fori_loop(0, NUM_FIELDS, field_row, 0)
    pltpu.sync_copy(out_v, out_hbm.at[pl.ds(wid * B_PER_W, B_PER_W)])


def _mlp_body(xe_ref, xc_ref, w1_ref, w2_ref, b_ref, o_ref):
    acc = jnp.dot(xe_ref[...], w1_ref[...], preferred_element_type=jnp.float32)
    acc = acc + jnp.dot(xc_ref[...], w2_ref[...], preferred_element_type=jnp.float32)
    o_ref[...] = jnp.maximum(acc + b_ref[...], 0.0)


def kernel(x_server_categorical, x_server_continuous, emb_tables, W, b):
    idx32 = x_server_categorical.astype(jnp.int32)
    flat_idx = idx32 + jnp.arange(NUM_FIELDS, dtype=jnp.int32) * VOCAB
    idx_wfb = flat_idx.reshape(NW, B_PER_W, NUM_FIELDS).transpose(0, 2, 1)
    table_pad = jnp.pad(emb_tables.reshape(N_ROWS, EMB_DIM),
                        ((0, 0), (0, PAD_D - EMB_DIM)))
    _ = _pad_table

    x_emb = _gather_sc(table_pad, idx_wfb)

    W1 = jnp.pad(W[:N_EMB].reshape(NUM_FIELDS, EMB_DIM, HIDDEN),
                 ((0, 0), (0, FIELD_W - EMB_DIM), (0, 0))
                 ).reshape(NUM_FIELDS * FIELD_W, HIDDEN)
    W2 = W[N_EMB:]
    b2d = b.reshape(1, HIDDEN)

    blk = 1024
    grid = BATCH // blk
    out = pl.pallas_call(
        _mlp_body,
        out_shape=jax.ShapeDtypeStruct((BATCH, HIDDEN), jnp.float32),
        grid=(grid,),
        in_specs=[
            pl.BlockSpec((blk, NUM_FIELDS * FIELD_W), lambda i: (i, 0)),
            pl.BlockSpec((blk, N_CONT), lambda i: (i, 0)),
            pl.BlockSpec((NUM_FIELDS * FIELD_W, HIDDEN), lambda i: (0, 0)),
            pl.BlockSpec((N_CONT, HIDDEN), lambda i: (0, 0)),
            pl.BlockSpec((1, HIDDEN), lambda i: (0, 0)),
        ],
        out_specs=pl.BlockSpec((blk, HIDDEN), lambda i: (i, 0)),
    )(x_emb, x_server_continuous, W1, W2, b2d)
    return out
